# baseline (speedup 1.0000x reference)
import functools
import numpy as np
import jax
from jax import lax
import jax.numpy as jnp
from jax.experimental import pallas as pl
from jax.experimental.pallas import tpu as pltpu
from jax.experimental.pallas import tpu_sc as plsc

_B = 128
_CC, _HH, _WW = 3, 224, 224
_NPOS = _CC * _HH * _WW
_ALPHA = 0.9
_BETA = 1.0 - _ALPHA

_NC, _NS, _L = 2, 16, 16
_NW = _NC * _NS
_PPW = _NPOS // _NW
_P = 168
_STEPS = _PPW // _P
_NG = _B // _L


_PERM = np.asarray([
    98, 105, 103, 43, 22, 94, 86, 125, 49, 0, 45, 108, 56, 121, 62, 109,
    3, 77, 9, 64, 5, 52, 50, 37, 78, 95, 30, 117, 127, 71, 53, 34,
    83, 18, 14, 116, 46, 1, 74, 124, 58, 92, 51, 81, 107, 48, 100, 42,
    106, 8, 69, 101, 90, 110, 66, 65, 21, 17, 67, 4, 32, 102, 27, 33,
    75, 89, 70, 123, 63, 104, 13, 39, 73, 85, 79, 120, 91, 41, 115, 6,
    59, 2, 57, 35, 99, 19, 40, 72, 118, 54, 80, 31, 126, 26, 97, 36,
    38, 25, 47, 61, 96, 15, 28, 68, 60, 82, 112, 55, 44, 119, 11, 114,
    10, 122, 76, 93, 84, 87, 16, 12, 88, 23, 29, 24, 7, 113, 111, 20,
], dtype=np.int32)

_PTAB = _PERM.reshape(_NG, _L)


def _axpy_chunk(pvecs, in_ref, o_ref):
    @plsc.parallel_loop(0, _P, unroll=4)
    def it(n):
        rown = jnp.full((_L,), n, jnp.int32)
        for k in range(_NG):
            sl = pl.ds(k * _L, _L)
            direct = in_ref[n, sl]
            mixed = plsc.load_gather(in_ref, [rown, pvecs[k]])
            o_ref[n, sl] = _ALPHA * direct + _BETA * mixed


def _sc_body(x_hbm, out_hbm,
             i0, i1, o0, o1, si0, si1, so0, so1):
    wid = lax.axis_index("s") * _NC + lax.axis_index("c")
    base = wid * _PPW
    ibufs, obufs = (i0, i1), (o0, o1)
    sis, sos = (si0, si1), (so0, so1)

    lane = lax.iota(jnp.int32, _L)
    pvecs = []
    for k in range(_NG):
        v = lane * 0
        for l in range(_L):
            v = jnp.where(lane == l, int(_PTAB[k, l]), v)
        pvecs.append(v)

    def src(s):
        return x_hbm.at[pl.ds(base + _P * s, _P)]

    def dst(s):
        return out_hbm.at[pl.ds(base + _P * s, _P)]

    for j in range(2):
        pltpu.make_async_copy(src(j), ibufs[j], sis[j]).start()

    def step(g, carry):
        for j in range(2):
            s = g * 2 + j
            pltpu.make_async_copy(src(s), ibufs[j], sis[j]).wait()

            @pl.when(s >= 2)
            def _():
                pltpu.make_async_copy(obufs[j], dst(s - 2), sos[j]).wait()

            _axpy_chunk(pvecs, ibufs[j], obufs[j])
            pltpu.make_async_copy(obufs[j], dst(s), sos[j]).start()

            @pl.when(s < _STEPS - 2)
            def _():
                pltpu.make_async_copy(src(s + 2), ibufs[j], sis[j]).start()
        return carry

    lax.fori_loop(0, _STEPS // 2, step, 0)

    for j in range(2):
        pltpu.make_async_copy(obufs[j], dst(_STEPS - 2 + j), sos[j]).wait()


@functools.partial(
    pl.kernel,
    out_type=jax.ShapeDtypeStruct((_NPOS, _B), jnp.float32),
    mesh=plsc.VectorSubcoreMesh(core_axis_name="c", subcore_axis_name="s"),
    compiler_params=pltpu.CompilerParams(needs_layout_passes=False),
    scratch_types=[
        pltpu.VMEM((_P, _B), jnp.float32),
        pltpu.VMEM((_P, _B), jnp.float32),
        pltpu.VMEM((_P, _B), jnp.float32),
        pltpu.VMEM((_P, _B), jnp.float32),
        pltpu.SemaphoreType.DMA,
        pltpu.SemaphoreType.DMA,
        pltpu.SemaphoreType.DMA,
        pltpu.SemaphoreType.DMA,
    ],
)
def _mixup_sc(x_hbm, out_hbm, *scratch):
    _sc_body(x_hbm, out_hbm, *scratch)


def kernel(x):
    xt = jnp.transpose(x, (1, 2, 3, 0)).reshape(_NPOS, _B)
    ot = _mixup_sc(xt)
    return ot.reshape(_CC, _HH, _WW, _B).transpose(3, 0, 1, 2)


# speedup vs baseline: 1.0226x; 1.0226x over previous
import functools
import numpy as np
import jax
from jax import lax
import jax.numpy as jnp
from jax.experimental import pallas as pl
from jax.experimental.pallas import tpu as pltpu
from jax.experimental.pallas import tpu_sc as plsc

_B = 128
_CC, _HH, _WW = 3, 224, 224
_NPOS = _CC * _HH * _WW
_ALPHA = 0.9
_BETA = 1.0 - _ALPHA

_NC, _NS, _L = 2, 16, 16
_NW = _NC * _NS
_PPW = _NPOS // _NW
_P = 168
_STEPS = _PPW // _P
_NG = _B // _L


_PERM = np.asarray([
    98, 105, 103, 43, 22, 94, 86, 125, 49, 0, 45, 108, 56, 121, 62, 109,
    3, 77, 9, 64, 5, 52, 50, 37, 78, 95, 30, 117, 127, 71, 53, 34,
    83, 18, 14, 116, 46, 1, 74, 124, 58, 92, 51, 81, 107, 48, 100, 42,
    106, 8, 69, 101, 90, 110, 66, 65, 21, 17, 67, 4, 32, 102, 27, 33,
    75, 89, 70, 123, 63, 104, 13, 39, 73, 85, 79, 120, 91, 41, 115, 6,
    59, 2, 57, 35, 99, 19, 40, 72, 118, 54, 80, 31, 126, 26, 97, 36,
    38, 25, 47, 61, 96, 15, 28, 68, 60, 82, 112, 55, 44, 119, 11, 114,
    10, 122, 76, 93, 84, 87, 16, 12, 88, 23, 29, 24, 7, 113, 111, 20,
], dtype=np.int32)

_PTAB = _PERM.reshape(_NG, _L)


def _axpy_chunk(pvecs, in_ref, o_ref):
    @plsc.parallel_loop(0, _P, unroll=2)
    def it(n):
        rown = jnp.full((_L,), n, jnp.int32)
        for k in range(_NG):
            sl = pl.ds(k * _L, _L)
            direct = in_ref[n, sl]
            mixed = plsc.load_gather(in_ref, [rown, pvecs[k]])
            o_ref[n, sl] = _ALPHA * direct + _BETA * mixed


def _sc_body(x_hbm, out_hbm,
             i0, i1, o0, o1, si0, si1, so0, so1):
    wid = lax.axis_index("s") * _NC + lax.axis_index("c")
    base = wid * _PPW
    ibufs, obufs = (i0, i1), (o0, o1)
    sis, sos = (si0, si1), (so0, so1)

    lane = lax.iota(jnp.int32, _L)
    pvecs = []
    for k in range(_NG):
        v = lane * 0
        for l in range(_L):
            v = jnp.where(lane == l, int(_PTAB[k, l]), v)
        pvecs.append(v)

    def src(s):
        return x_hbm.at[pl.ds(base + _P * s, _P)]

    def dst(s):
        return out_hbm.at[pl.ds(base + _P * s, _P)]

    for j in range(2):
        pltpu.make_async_copy(src(j), ibufs[j], sis[j]).start()

    def step(g, carry):
        for j in range(2):
            s = g * 2 + j
            pltpu.make_async_copy(src(s), ibufs[j], sis[j]).wait()

            @pl.when(s >= 2)
            def _():
                pltpu.make_async_copy(obufs[j], dst(s - 2), sos[j]).wait()

            _axpy_chunk(pvecs, ibufs[j], obufs[j])
            pltpu.make_async_copy(obufs[j], dst(s), sos[j]).start()

            @pl.when(s < _STEPS - 2)
            def _():
                pltpu.make_async_copy(src(s + 2), ibufs[j], sis[j]).start()
        return carry

    lax.fori_loop(0, _STEPS // 2, step, 0)

    for j in range(2):
        pltpu.make_async_copy(obufs[j], dst(_STEPS - 2 + j), sos[j]).wait()


@functools.partial(
    pl.kernel,
    out_type=jax.ShapeDtypeStruct((_NPOS, _B), jnp.float32),
    mesh=plsc.VectorSubcoreMesh(core_axis_name="c", subcore_axis_name="s"),
    compiler_params=pltpu.CompilerParams(needs_layout_passes=False),
    scratch_types=[
        pltpu.VMEM((_P, _B), jnp.float32),
        pltpu.VMEM((_P, _B), jnp.float32),
        pltpu.VMEM((_P, _B), jnp.float32),
        pltpu.VMEM((_P, _B), jnp.float32),
        pltpu.SemaphoreType.DMA,
        pltpu.SemaphoreType.DMA,
        pltpu.SemaphoreType.DMA,
        pltpu.SemaphoreType.DMA,
    ],
)
def _mixup_sc(x_hbm, out_hbm, *scratch):
    _sc_body(x_hbm, out_hbm, *scratch)


def kernel(x):
    xt = jnp.transpose(x, (1, 2, 3, 0)).reshape(_NPOS, _B)
    ot = _mixup_sc(xt)
    return ot.reshape(_CC, _HH, _WW, _B).transpose(3, 0, 1, 2)
